# baseline (speedup 1.0000x reference)
import functools

import jax
import jax.numpy as jnp
from jax import lax
from jax.experimental import pallas as pl
from jax.experimental.pallas import tpu as pltpu
from jax.experimental.pallas import tpu_sc as plsc

NC = 2
NS = 16
NW = NC * NS
CH = 80
NBUF = 5
DEPTH = 3


def _make_gather(ntok, hidden):
    per_w = ntok // NW
    nch = per_w // CH
    assert nch % NBUF == 0
    mesh = plsc.VectorSubcoreMesh(core_axis_name="c", subcore_axis_name="s")

    @functools.partial(
        pl.kernel,
        mesh=mesh,
        out_type=jax.ShapeDtypeStruct((ntok, hidden), jnp.float32),
        scratch_types=[
            pltpu.VMEM((nch, CH), jnp.int32),
        ]
        + [pltpu.VMEM((CH, hidden), jnp.float32) for _ in range(NBUF)]
        + [pltpu.SemaphoreType.DMA for _ in range(2 * NBUF)],
    )
    def gather_rows(table_hbm, idx_hbm, out_hbm, idx_v, *bufs_and_sems):
        bufs = bufs_and_sems[:NBUF]
        semg = bufs_and_sems[NBUF:2 * NBUF]
        semw = bufs_and_sems[2 * NBUF:]
        wid = lax.axis_index("s") * NC + lax.axis_index("c")
        base = wid * per_w
        pltpu.sync_copy(idx_hbm.at[wid], idx_v)

        def start_gather(k, slot):
            pltpu.async_copy(table_hbm.at[idx_v.at[k]], bufs[slot], semg[slot])

        def wait_gather(k, slot):
            pltpu.make_async_copy(table_hbm.at[idx_v.at[k]], bufs[slot],
                                  semg[slot]).wait()

        def start_write(k, slot):
            pltpu.async_copy(bufs[slot], out_hbm.at[pl.ds(base + k * CH, CH)],
                             semw[slot])

        def wait_write(k, slot):
            pltpu.make_async_copy(bufs[slot],
                                  out_hbm.at[pl.ds(base + k * CH, CH)],
                                  semw[slot]).wait()

        for k in range(DEPTH):
            start_gather(k, k)

        def body(j, carry):
            for b in range(NBUF):
                i = j * NBUF + b
                k = i + DEPTH
                kb = (b + DEPTH) % NBUF

                @pl.when(k < nch)
                def _():
                    @pl.when(k >= NBUF)
                    def _():
                        wait_write(k - NBUF, kb)

                    start_gather(k, kb)

                wait_gather(i, b)
                start_write(i, b)
            return carry

        lax.fori_loop(0, nch // NBUF, body, 0)
        for b in range(NBUF):
            wait_write(nch - NBUF + b, b)

    return gather_rows


def _ln_body(we_ref, tt_ref, pete_ref, dte_ref, g_ref, b_ref, out_ref):
    bb, l, h = we_ref.shape
    x = we_ref[...]
    _ln_compute(x, tt_ref, pete_ref, dte_ref, g_ref, b_ref, out_ref)


def _ln_body_aliased(prev_ref, we_ref, tt_ref, pete_ref, dte_ref, g_ref,
                     b_ref, out_ref):
    del prev_ref
    x = we_ref[...]
    _ln_compute(x, tt_ref, pete_ref, dte_ref, g_ref, b_ref, out_ref)


def _ln_compute(x, tt_ref, pete_ref, dte_ref, g_ref, b_ref, out_ref):
    tt = tt_ref[...]
    ttf = lax.broadcast_in_dim(tt.astype(jnp.float32), x.shape, (0, 1))
    x = x + pete_ref[...][None, :, :] + ttf * dte_ref[...][None, :, :]
    mean = jnp.mean(x, axis=-1, keepdims=True)
    var = jnp.mean(jnp.square(x - mean), axis=-1, keepdims=True)
    y = (x - mean) * lax.rsqrt(var + 1e-5)
    out_ref[...] = y * g_ref[...][None, :, :] + b_ref[...][None, :, :]


def kernel(input_ids, token_type_ids, word_emb, pos_emb, type_emb, ln_gamma, ln_beta):
    b, l = input_ids.shape
    hidden = word_emb.shape[1]
    ntok = b * l

    ids = input_ids.reshape(ntok).astype(jnp.int32)
    bks = (128, 384, 384, 128)
    K = len(bks)
    starts = [sum(bks[:k]) for k in range(K)]
    gathers = {bk: _make_gather(bk * l, hidden) for bk in set(bks)}
    ids3 = ids.reshape(b, l)
    wek = []
    for k, bk in enumerate(bks):
        idsk = lax.dynamic_slice_in_dim(ids3, starts[k], bk, 0).reshape(
            NW, bk * l // (NW * CH), CH)
        wek.append(gathers[bk](word_emb, idsk))

    tt = token_type_ids.astype(jnp.int32)
    pete = pos_emb[:l] + type_emb[0]
    dte = (type_emb[1] - type_emb[0]).reshape(1, hidden)
    g = ln_gamma.reshape(1, hidden)
    beta = ln_beta.reshape(1, hidden)

    bb = 64
    small_specs = [
        pl.BlockSpec((l, hidden), lambda i: (0, 0)),
        pl.BlockSpec((1, hidden), lambda i: (0, 0)),
        pl.BlockSpec((1, hidden), lambda i: (0, 0)),
        pl.BlockSpec((1, hidden), lambda i: (0, 0)),
    ]
    out = None
    for k, bk in enumerate(bks):
        we3k = wek[k].reshape(bk, l, hidden)
        nb0 = starts[k] // bb
        chunk_specs = [
            pl.BlockSpec((bb, l, hidden), lambda i: (i, 0, 0)),
            pl.BlockSpec((bb, l), lambda i, nb0=nb0: (nb0 + i, 0)),
        ] + small_specs
        out_spec = pl.BlockSpec((bb, l, hidden),
                                lambda i, nb0=nb0: (nb0 + i, 0, 0))
        common = dict(
            grid=(bk // bb,),
            out_specs=out_spec,
            out_shape=jax.ShapeDtypeStruct((b, l, hidden), jnp.float32),
        )
        if k == 0:
            out = pl.pallas_call(_ln_body, in_specs=chunk_specs, **common)(
                we3k, tt, pete, dte, g, beta)
        else:
            out = pl.pallas_call(
                _ln_body_aliased,
                in_specs=[pl.BlockSpec(memory_space=pltpu.MemorySpace.HBM)] + chunk_specs,
                input_output_aliases={0: 0},
                **common)(out, we3k, tt, pete, dte, g, beta)
    return out


# speedup vs baseline: 1.0098x; 1.0098x over previous
import functools

import jax
import jax.numpy as jnp
from jax import lax
from jax.experimental import pallas as pl
from jax.experimental.pallas import tpu as pltpu
from jax.experimental.pallas import tpu_sc as plsc

NC = 2
NS = 16
NW = NC * NS
CH = 80
NBUF = 5
DEPTH = 3


def _make_gather(ntok, hidden):
    per_w = ntok // NW
    nch = per_w // CH
    assert nch % NBUF == 0
    mesh = plsc.VectorSubcoreMesh(core_axis_name="c", subcore_axis_name="s")

    @functools.partial(
        pl.kernel,
        mesh=mesh,
        out_type=jax.ShapeDtypeStruct((ntok, hidden), jnp.float32),
        scratch_types=[
            pltpu.VMEM((nch, CH), jnp.int32),
        ]
        + [pltpu.VMEM((CH, hidden), jnp.float32) for _ in range(NBUF)]
        + [pltpu.SemaphoreType.DMA for _ in range(2 * NBUF)],
    )
    def gather_rows(table_hbm, idx_hbm, out_hbm, idx_v, *bufs_and_sems):
        bufs = bufs_and_sems[:NBUF]
        semg = bufs_and_sems[NBUF:2 * NBUF]
        semw = bufs_and_sems[2 * NBUF:]
        wid = lax.axis_index("s") * NC + lax.axis_index("c")
        base = wid * per_w
        pltpu.sync_copy(idx_hbm.at[wid], idx_v)

        def start_gather(k, slot):
            pltpu.async_copy(table_hbm.at[idx_v.at[k]], bufs[slot], semg[slot])

        def wait_gather(k, slot):
            pltpu.make_async_copy(table_hbm.at[idx_v.at[k]], bufs[slot],
                                  semg[slot]).wait()

        def start_write(k, slot):
            pltpu.async_copy(bufs[slot], out_hbm.at[pl.ds(base + k * CH, CH)],
                             semw[slot])

        def wait_write(k, slot):
            pltpu.make_async_copy(bufs[slot],
                                  out_hbm.at[pl.ds(base + k * CH, CH)],
                                  semw[slot]).wait()

        for k in range(DEPTH):
            start_gather(k, k)

        def body(j, carry):
            for b in range(NBUF):
                i = j * NBUF + b
                k = i + DEPTH
                kb = (b + DEPTH) % NBUF

                @pl.when(k < nch)
                def _():
                    @pl.when(k >= NBUF)
                    def _():
                        wait_write(k - NBUF, kb)

                    start_gather(k, kb)

                wait_gather(i, b)
                start_write(i, b)
            return carry

        lax.fori_loop(0, nch // NBUF, body, 0)
        for b in range(NBUF):
            wait_write(nch - NBUF + b, b)

    return gather_rows


def _ln_body(we_ref, tt_ref, pete_ref, dte_ref, g_ref, b_ref, out_ref):
    bb, l, h = we_ref.shape
    x = we_ref[...]
    _ln_compute(x, tt_ref, pete_ref, dte_ref, g_ref, b_ref, out_ref)


def _ln_body_aliased(prev_ref, we_ref, tt_ref, pete_ref, dte_ref, g_ref,
                     b_ref, out_ref):
    del prev_ref
    x = we_ref[...]
    _ln_compute(x, tt_ref, pete_ref, dte_ref, g_ref, b_ref, out_ref)


def _ln_compute(x, tt_ref, pete_ref, dte_ref, g_ref, b_ref, out_ref):
    tt = tt_ref[...]
    ttf = lax.broadcast_in_dim(tt.astype(jnp.float32), x.shape, (0, 1))
    x = x + pete_ref[...][None, :, :] + ttf * dte_ref[...][None, :, :]
    mean = jnp.mean(x, axis=-1, keepdims=True)
    var = jnp.mean(jnp.square(x - mean), axis=-1, keepdims=True)
    y = (x - mean) * lax.rsqrt(var + 1e-5)
    out_ref[...] = y * g_ref[...][None, :, :] + b_ref[...][None, :, :]


def kernel(input_ids, token_type_ids, word_emb, pos_emb, type_emb, ln_gamma, ln_beta):
    b, l = input_ids.shape
    hidden = word_emb.shape[1]
    ntok = b * l

    ids = input_ids.reshape(ntok).astype(jnp.int32)
    bks = (64, 320, 320, 320)
    K = len(bks)
    starts = [sum(bks[:k]) for k in range(K)]
    gathers = {bk: _make_gather(bk * l, hidden) for bk in set(bks)}
    ids3 = ids.reshape(b, l)
    wek = []
    for k, bk in enumerate(bks):
        idsk = lax.dynamic_slice_in_dim(ids3, starts[k], bk, 0).reshape(
            NW, bk * l // (NW * CH), CH)
        wek.append(gathers[bk](word_emb, idsk))

    tt = token_type_ids.astype(jnp.int32)
    pete = pos_emb[:l] + type_emb[0]
    dte = (type_emb[1] - type_emb[0]).reshape(1, hidden)
    g = ln_gamma.reshape(1, hidden)
    beta = ln_beta.reshape(1, hidden)

    bb = 64
    small_specs = [
        pl.BlockSpec((l, hidden), lambda i: (0, 0)),
        pl.BlockSpec((1, hidden), lambda i: (0, 0)),
        pl.BlockSpec((1, hidden), lambda i: (0, 0)),
        pl.BlockSpec((1, hidden), lambda i: (0, 0)),
    ]
    out = None
    for k, bk in enumerate(bks):
        we3k = wek[k].reshape(bk, l, hidden)
        nb0 = starts[k] // bb
        chunk_specs = [
            pl.BlockSpec((bb, l, hidden), lambda i: (i, 0, 0)),
            pl.BlockSpec((bb, l), lambda i, nb0=nb0: (nb0 + i, 0)),
        ] + small_specs
        out_spec = pl.BlockSpec((bb, l, hidden),
                                lambda i, nb0=nb0: (nb0 + i, 0, 0))
        common = dict(
            grid=(bk // bb,),
            out_specs=out_spec,
            out_shape=jax.ShapeDtypeStruct((b, l, hidden), jnp.float32),
        )
        if k == 0:
            out = pl.pallas_call(_ln_body, in_specs=chunk_specs, **common)(
                we3k, tt, pete, dte, g, beta)
        else:
            out = pl.pallas_call(
                _ln_body_aliased,
                in_specs=[pl.BlockSpec(memory_space=pltpu.MemorySpace.HBM)] + chunk_specs,
                input_output_aliases={0: 0},
                **common)(out, we3k, tt, pete, dte, g, beta)
    return out


# speedup vs baseline: 1.0210x; 1.0111x over previous
import functools

import jax
import jax.numpy as jnp
from jax import lax
from jax.experimental import pallas as pl
from jax.experimental.pallas import tpu as pltpu
from jax.experimental.pallas import tpu_sc as plsc

NC = 2
NS = 16
NW = NC * NS
CH = 64
NBUF = 5
DEPTH = 3


def _make_gather(ntok, hidden):
    per_w = ntok // NW
    nch = per_w // CH
    assert nch % NBUF == 0
    mesh = plsc.VectorSubcoreMesh(core_axis_name="c", subcore_axis_name="s")

    @functools.partial(
        pl.kernel,
        mesh=mesh,
        out_type=jax.ShapeDtypeStruct((ntok, hidden), jnp.float32),
        scratch_types=[
            pltpu.VMEM((nch, CH), jnp.int32),
        ]
        + [pltpu.VMEM((CH, hidden), jnp.float32) for _ in range(NBUF)]
        + [pltpu.SemaphoreType.DMA for _ in range(2 * NBUF)],
    )
    def gather_rows(table_hbm, idx_hbm, out_hbm, idx_v, *bufs_and_sems):
        bufs = bufs_and_sems[:NBUF]
        semg = bufs_and_sems[NBUF:2 * NBUF]
        semw = bufs_and_sems[2 * NBUF:]
        wid = lax.axis_index("s") * NC + lax.axis_index("c")
        base = wid * per_w
        pltpu.sync_copy(idx_hbm.at[wid], idx_v)

        def start_gather(k, slot):
            pltpu.async_copy(table_hbm.at[idx_v.at[k]], bufs[slot], semg[slot])

        def wait_gather(k, slot):
            pltpu.make_async_copy(table_hbm.at[idx_v.at[k]], bufs[slot],
                                  semg[slot]).wait()

        def start_write(k, slot):
            pltpu.async_copy(bufs[slot], out_hbm.at[pl.ds(base + k * CH, CH)],
                             semw[slot])

        def wait_write(k, slot):
            pltpu.make_async_copy(bufs[slot],
                                  out_hbm.at[pl.ds(base + k * CH, CH)],
                                  semw[slot]).wait()

        for k in range(DEPTH):
            start_gather(k, k)

        def body(j, carry):
            for b in range(NBUF):
                i = j * NBUF + b
                k = i + DEPTH
                kb = (b + DEPTH) % NBUF

                @pl.when(k < nch)
                def _():
                    @pl.when(k >= NBUF)
                    def _():
                        wait_write(k - NBUF, kb)

                    start_gather(k, kb)

                wait_gather(i, b)
                start_write(i, b)
            return carry

        lax.fori_loop(0, nch // NBUF, body, 0)
        for b in range(NBUF):
            wait_write(nch - NBUF + b, b)

    return gather_rows


def _ln_body(we_ref, tt_ref, pete_ref, dte_ref, g_ref, b_ref, out_ref):
    bb, l, h = we_ref.shape
    x = we_ref[...]
    _ln_compute(x, tt_ref, pete_ref, dte_ref, g_ref, b_ref, out_ref)


def _ln_body_aliased(prev_ref, we_ref, tt_ref, pete_ref, dte_ref, g_ref,
                     b_ref, out_ref):
    del prev_ref
    x = we_ref[...]
    _ln_compute(x, tt_ref, pete_ref, dte_ref, g_ref, b_ref, out_ref)


def _ln_compute(x, tt_ref, pete_ref, dte_ref, g_ref, b_ref, out_ref):
    tt = tt_ref[...]
    ttf = lax.broadcast_in_dim(tt.astype(jnp.float32), x.shape, (0, 1))
    x = x + pete_ref[...][None, :, :] + ttf * dte_ref[...][None, :, :]
    mean = jnp.mean(x, axis=-1, keepdims=True)
    var = jnp.mean(jnp.square(x - mean), axis=-1, keepdims=True)
    y = (x - mean) * lax.rsqrt(var + 1e-5)
    out_ref[...] = y * g_ref[...][None, :, :] + b_ref[...][None, :, :]


def kernel(input_ids, token_type_ids, word_emb, pos_emb, type_emb, ln_gamma, ln_beta):
    b, l = input_ids.shape
    hidden = word_emb.shape[1]
    ntok = b * l

    ids = input_ids.reshape(ntok).astype(jnp.int32)
    bks = (256, 256, 256, 256)
    K = len(bks)
    starts = [sum(bks[:k]) for k in range(K)]
    gathers = {bk: _make_gather(bk * l, hidden) for bk in set(bks)}
    ids3 = ids.reshape(b, l)
    wek = []
    for k, bk in enumerate(bks):
        idsk = lax.dynamic_slice_in_dim(ids3, starts[k], bk, 0).reshape(
            NW, bk * l // (NW * CH), CH)
        wek.append(gathers[bk](word_emb, idsk))

    tt = token_type_ids.astype(jnp.int32)
    pete = pos_emb[:l] + type_emb[0]
    dte = (type_emb[1] - type_emb[0]).reshape(1, hidden)
    g = ln_gamma.reshape(1, hidden)
    beta = ln_beta.reshape(1, hidden)

    bb = 64
    small_specs = [
        pl.BlockSpec((l, hidden), lambda i: (0, 0)),
        pl.BlockSpec((1, hidden), lambda i: (0, 0)),
        pl.BlockSpec((1, hidden), lambda i: (0, 0)),
        pl.BlockSpec((1, hidden), lambda i: (0, 0)),
    ]
    out = None
    for k, bk in enumerate(bks):
        we3k = wek[k].reshape(bk, l, hidden)
        nb0 = starts[k] // bb
        chunk_specs = [
            pl.BlockSpec((bb, l, hidden), lambda i: (i, 0, 0)),
            pl.BlockSpec((bb, l), lambda i, nb0=nb0: (nb0 + i, 0)),
        ] + small_specs
        out_spec = pl.BlockSpec((bb, l, hidden),
                                lambda i, nb0=nb0: (nb0 + i, 0, 0))
        common = dict(
            grid=(bk // bb,),
            out_specs=out_spec,
            out_shape=jax.ShapeDtypeStruct((b, l, hidden), jnp.float32),
        )
        if k == 0:
            out = pl.pallas_call(_ln_body, in_specs=chunk_specs, **common)(
                we3k, tt, pete, dte, g, beta)
        else:
            out = pl.pallas_call(
                _ln_body_aliased,
                in_specs=[pl.BlockSpec(memory_space=pltpu.MemorySpace.HBM)] + chunk_specs,
                input_output_aliases={0: 0},
                **common)(out, we3k, tt, pete, dte, g, beta)
    return out
